# baseline (speedup 1.0000x reference)
import functools

import jax
import jax.numpy as jnp
from jax.experimental import pallas as pl

K_NBRS = 20
K_NORMAL = 10
KW = 1.0
REG = 1e-3


def _mm_relu_body(x_ref, w_ref, b_ref, o_ref):
    o_ref[...] = jax.nn.relu(
        jnp.dot(x_ref[...], w_ref[...], preferred_element_type=jnp.float32)
        + b_ref[...]
    )


def _mm_relu(x, w, b):
    n, _ = x.shape
    co = w.shape[1]
    return pl.pallas_call(
        _mm_relu_body,
        out_shape=jax.ShapeDtypeStruct((n, co), jnp.float32),
    )(x, w, b[None, :])


def _knn(pos, k):
    sq = jnp.sum(pos ** 2, axis=1)
    d = sq[:, None] + sq[None, :] - 2.0 * (pos @ pos.T)
    _, idx = jax.lax.top_k(-d, k)
    return idx


def _basis(pos, idx):
    nb = pos[idx]
    cen = nb - nb.mean(axis=1, keepdims=True)
    cov = jnp.einsum('nki,nkj->nij', cen, cen)
    _, V = jnp.linalg.eigh(cov)
    n = V[:, :, 0]
    sgn = jnp.where(jnp.sum(n * pos, axis=-1, keepdims=True) < 0, -1.0, 1.0)
    n = n * sgn
    xb = V[:, :, 2]
    yb = jnp.cross(n, xb)
    return n, xb, yb


def _grad_ls(pos, xb, yb, idx, kw, reg):
    off = pos[idx] - pos[:, None, :]
    X = jnp.stack([jnp.einsum('nkd,nd->nk', off, xb),
                   jnp.einsum('nkd,nd->nk', off, yb)], axis=-1)
    w = jnp.exp(-jnp.sum(off ** 2, axis=-1) / (kw ** 2))
    A = jnp.einsum('nka,nk,nkb->nab', X, w, X) + reg * jnp.eye(2, dtype=pos.dtype)
    B = jnp.einsum('nka,nk->nak', X, w)
    G = jnp.linalg.solve(A, B)
    return G


def _grad_op(G, idx, s):
    diff = s[idx] - s[:, None, :]
    return jnp.einsum('nak,nkc->nac', G, diff)


def _div_op(G, idx, v, n):
    c = v.shape[-1]
    contrib = jnp.einsum('nak,nac->nkc', G, v)
    d = jnp.zeros((n, c), v.dtype).at[idx.reshape(-1)].add(contrib.reshape(-1, c))
    d = d.at[jnp.arange(n)].add(-jnp.einsum('nak,nac->nc', G, v))
    return d


def _delta_conv(x, v, G, idx, n, Ws, bs, Wv):
    vn = jnp.sqrt(jnp.sum(v ** 2, axis=1) + 1e-8)
    dv = _div_op(G, idx, v, n)
    h = _mm_relu(jnp.concatenate([x, vn, dv], axis=-1), Ws, bs)
    if Wv is None:
        return h, None
    v_new = jnp.einsum('nac,co->nao', v, Wv) + _grad_op(G, idx, h)
    return h, v_new


def kernel(points, x1_dummy, als_ppoints, Ws1, bs1, Wv1, Ws2, bs2, Wv2, Ws3, bs3):
    pos = jnp.concatenate([points, als_ppoints], axis=0)
    n = pos.shape[0]
    pos_sg = jax.lax.stop_gradient(pos)
    idx = _knn(pos_sg, K_NBRS)
    idx_n = _knn(pos_sg, K_NORMAL)
    _, xb, yb = _basis(pos_sg, idx_n)
    G = _grad_ls(pos_sg, xb, yb, idx, KW, REG)
    x = pos - pos.mean(axis=0, keepdims=True)
    v = _grad_op(G, idx, pos)
    out = []
    x, v = _delta_conv(x, v, G, idx, n, Ws1, bs1, Wv1)
    out.append(x)
    x, v = _delta_conv(x, v, G, idx, n, Ws2, bs2, Wv2)
    out.append(x)
    x, _ = _delta_conv(x, v, G, idx, n, Ws3, bs3, None)
    out.append(x)
    return tuple(out)


# speedup vs baseline: 9.2271x; 9.2271x over previous
import functools

import jax
import jax.numpy as jnp
from jax.experimental import pallas as pl

K_NBRS = 20
K_NORMAL = 10
KW = 1.0
REG = 1e-3

_BQ = 256
_BN = 1024


def _knn_body(aq_ref, bk_ref, sqc_ref, sqr_ref, idx_ref, *, k):
    m = jax.lax.dot_general(
        aq_ref[...], bk_ref[...], (((1,), (0,)), ((), ())),
        preferred_element_type=jnp.float32)
    vals = (sqc_ref[...] + sqr_ref[...]) - 2.0 * m
    iota = jax.lax.broadcasted_iota(jnp.int32, vals.shape, 1)
    big = jnp.int32(2 ** 30)
    cols = []
    for _ in range(k):
        mn = jnp.min(vals, axis=1, keepdims=True)
        am = jnp.min(jnp.where(vals <= mn, iota, big), axis=1, keepdims=True)
        cols.append(am)
        vals = jnp.where(iota == am, jnp.inf, vals)
    idx_ref[...] = jnp.concatenate(cols, axis=1)


def _knn_pallas(pos, k):
    n = pos.shape[0]
    sq = jnp.sum(pos * pos, axis=1)
    a_q = jnp.concatenate([pos, jnp.zeros((n, 5), jnp.float32)], axis=1)
    b_k = jnp.concatenate([pos.T, jnp.zeros((5, n), jnp.float32)], axis=0)
    return pl.pallas_call(
        functools.partial(_knn_body, k=k),
        grid=(n // _BQ,),
        in_specs=[
            pl.BlockSpec((_BQ, 8), lambda i: (i, 0)),
            pl.BlockSpec((8, n), lambda i: (0, 0)),
            pl.BlockSpec((_BQ, 1), lambda i: (i, 0)),
            pl.BlockSpec((1, n), lambda i: (0, 0)),
        ],
        out_specs=pl.BlockSpec((_BQ, k), lambda i: (i, 0)),
        out_shape=jax.ShapeDtypeStruct((n, k), jnp.int32),
    )(a_q, b_k, sq[:, None], sq[None, :])


def _jacobi_rot(app, aqq, apq):
    zero = apq == 0.0
    tau = (aqq - app) / (2.0 * jnp.where(zero, 1.0, apq))
    t = jnp.sign(tau) / (jnp.abs(tau) + jnp.sqrt(1.0 + tau * tau))
    t = jnp.where(jnp.isnan(t), 0.0, t)
    c = jax.lax.rsqrt(1.0 + t * t)
    s = t * c
    c = jnp.where(zero, 1.0, c)
    s = jnp.where(zero, 0.0, s)
    return c, s


def _smallest_evec(a11, a22, a33, a12, a13, a23):
    v = [[jnp.ones_like(a11), jnp.zeros_like(a11), jnp.zeros_like(a11)],
         [jnp.zeros_like(a11), jnp.ones_like(a11), jnp.zeros_like(a11)],
         [jnp.zeros_like(a11), jnp.zeros_like(a11), jnp.ones_like(a11)]]
    m = {(0, 0): a11, (1, 1): a22, (2, 2): a33,
         (0, 1): a12, (0, 2): a13, (1, 2): a23}
    for _ in range(6):
        for (p, q) in ((0, 1), (0, 2), (1, 2)):
            r = 3 - p - q
            c, s = _jacobi_rot(m[(p, p)], m[(q, q)], m[(p, q)])
            app, aqq, apq = m[(p, p)], m[(q, q)], m[(p, q)]
            apr = m[(min(p, r), max(p, r))]
            aqr = m[(min(q, r), max(q, r))]
            m[(p, p)] = c * c * app - 2.0 * s * c * apq + s * s * aqq
            m[(q, q)] = s * s * app + 2.0 * s * c * apq + c * c * aqq
            m[(p, q)] = jnp.zeros_like(apq)
            m[(min(p, r), max(p, r))] = c * apr - s * aqr
            m[(min(q, r), max(q, r))] = s * apr + c * aqr
            for i in range(3):
                vip, viq = v[i][p], v[i][q]
                v[i][p] = c * vip - s * viq
                v[i][q] = s * vip + c * viq
    d = [m[(0, 0)], m[(1, 1)], m[(2, 2)]]
    c1 = (d[0] <= d[1]) & (d[0] <= d[2])
    c2 = (d[1] <= d[0]) & (d[1] <= d[2])
    pick = lambda row: jnp.where(c1, row[0], jnp.where(c2, row[1], row[2]))
    nx, ny, nz = pick(v[0]), pick(v[1]), pick(v[2])
    inv = jax.lax.rsqrt(nx * nx + ny * ny + nz * nz)
    return nx * inv, ny * inv, nz * inv


def _geom_body(off_ref, g_ref, v0_ref):
    off = off_ref[...]
    ox, oy, oz = off[0], off[1], off[2]
    cx = ox[:K_NORMAL] - jnp.mean(ox[:K_NORMAL], axis=0, keepdims=True)
    cy = oy[:K_NORMAL] - jnp.mean(oy[:K_NORMAL], axis=0, keepdims=True)
    cz = oz[:K_NORMAL] - jnp.mean(oz[:K_NORMAL], axis=0, keepdims=True)
    nx, ny, nz = _smallest_evec(
        jnp.sum(cx * cx, axis=0), jnp.sum(cy * cy, axis=0),
        jnp.sum(cz * cz, axis=0), jnp.sum(cx * cy, axis=0),
        jnp.sum(cx * cz, axis=0), jnp.sum(cy * cz, axis=0))
    cnd = jnp.abs(nz) < 0.9
    tx = jnp.where(cnd, ny, 0.0)
    ty = jnp.where(cnd, -nx, nz)
    tz = jnp.where(cnd, 0.0, -ny)
    tinv = jax.lax.rsqrt(tx * tx + ty * ty + tz * tz)
    xbx, xby, xbz = tx * tinv, ty * tinv, tz * tinv
    ybx = ny * xbz - nz * xby
    yby = nz * xbx - nx * xbz
    ybz = nx * xby - ny * xbx
    x1 = ox * xbx + oy * xby + oz * xbz
    x2 = ox * ybx + oy * yby + oz * ybz
    w = jnp.exp(-(ox * ox + oy * oy + oz * oz) / (KW * KW))
    a11 = jnp.sum(w * x1 * x1, axis=0) + REG
    a22 = jnp.sum(w * x2 * x2, axis=0) + REG
    a12 = jnp.sum(w * x1 * x2, axis=0)
    dinv = 1.0 / (a11 * a22 - a12 * a12)
    i11, i12, i22 = a22 * dinv, -a12 * dinv, a11 * dinv
    g1 = (i11 * x1 + i12 * x2) * w
    g2 = (i12 * x1 + i22 * x2) * w
    g_ref[...] = jnp.stack([g1, g2], axis=0)
    v0_ref[...] = jnp.stack(
        [jnp.stack([jnp.sum(g1 * ox, axis=0), jnp.sum(g1 * oy, axis=0),
                    jnp.sum(g1 * oz, axis=0)], axis=0),
         jnp.stack([jnp.sum(g2 * ox, axis=0), jnp.sum(g2 * oy, axis=0),
                    jnp.sum(g2 * oz, axis=0)], axis=0)], axis=0)


def _geom_pallas(off_t):
    n = off_t.shape[2]
    return pl.pallas_call(
        _geom_body,
        grid=(n // _BN,),
        in_specs=[pl.BlockSpec((3, K_NBRS, _BN), lambda i: (0, 0, i))],
        out_specs=[
            pl.BlockSpec((2, K_NBRS, _BN), lambda i: (0, 0, i)),
            pl.BlockSpec((2, 3, _BN), lambda i: (0, 0, i)),
        ],
        out_shape=[
            jax.ShapeDtypeStruct((2, K_NBRS, n), jnp.float32),
            jax.ShapeDtypeStruct((2, 3, n), jnp.float32),
        ],
    )(off_t)


def _mm_relu_body(x_ref, w_ref, b_ref, o_ref):
    o_ref[...] = jax.nn.relu(
        jnp.dot(x_ref[...], w_ref[...], preferred_element_type=jnp.float32)
        + b_ref[...])


def _mm_relu(x, w, b):
    n, _ = x.shape
    co = w.shape[1]
    return pl.pallas_call(
        _mm_relu_body,
        out_shape=jax.ShapeDtypeStruct((n, co), jnp.float32),
    )(x, w, b[None, :])


def _grad_op(G, idx, s):
    diff = s[idx] - s[:, None, :]
    return jnp.einsum('nak,nkc->nac', G, diff)


def _div_op(G, idx, v, n):
    c = v.shape[-1]
    contrib = jnp.einsum('nak,nac->nkc', G, v)
    d = jnp.zeros((n, c), v.dtype).at[idx.reshape(-1)].add(contrib.reshape(-1, c))
    d = d.at[jnp.arange(n)].add(-jnp.einsum('nak,nac->nc', G, v))
    return d


def _delta_conv(x, v, G, idx, n, Ws, bs, Wv):
    vn = jnp.sqrt(jnp.sum(v ** 2, axis=1) + 1e-8)
    dv = _div_op(G, idx, v, n)
    h = _mm_relu(jnp.concatenate([x, vn, dv], axis=-1), Ws, bs)
    if Wv is None:
        return h, None
    v_new = jnp.einsum('nac,co->nao', v, Wv) + _grad_op(G, idx, h)
    return h, v_new


def kernel(points, x1_dummy, als_ppoints, Ws1, bs1, Wv1, Ws2, bs2, Wv2, Ws3, bs3):
    pos = jnp.concatenate([points, als_ppoints], axis=0)
    n = pos.shape[0]
    idx = _knn_pallas(pos, K_NBRS)
    off = pos[idx] - pos[:, None, :]
    g_t, v0_t = _geom_pallas(off.transpose(2, 1, 0))
    G = g_t.transpose(2, 0, 1)
    v = v0_t.transpose(2, 0, 1)
    x = pos - pos.mean(axis=0, keepdims=True)
    out = []
    x, v = _delta_conv(x, v, G, idx, n, Ws1, bs1, Wv1)
    out.append(x)
    x, v = _delta_conv(x, v, G, idx, n, Ws2, bs2, Wv2)
    out.append(x)
    x, _ = _delta_conv(x, v, G, idx, n, Ws3, bs3, None)
    out.append(x)
    return tuple(out)


# speedup vs baseline: 16.0668x; 1.7413x over previous
import functools

import jax
import jax.numpy as jnp
from jax import lax
from jax.experimental import pallas as pl
from jax.experimental.pallas import tpu as pltpu
from jax.experimental.pallas import tpu_sc as plsc

K_NBRS = 20
K_NORMAL = 10
KW = 1.0
REG = 1e-3

_BQ = 256
_BN = 1024


def _knn_body(aq_ref, bk_ref, sqc_ref, sqr_ref, idx_ref, *, k):
    m = jax.lax.dot_general(
        aq_ref[...], bk_ref[...], (((1,), (0,)), ((), ())),
        preferred_element_type=jnp.float32)
    vals = (sqc_ref[...] + sqr_ref[...]) - 2.0 * m
    iota = jax.lax.broadcasted_iota(jnp.int32, vals.shape, 1)
    big = jnp.int32(2 ** 30)
    cols = []
    for _ in range(k):
        mn = jnp.min(vals, axis=1, keepdims=True)
        am = jnp.min(jnp.where(vals <= mn, iota, big), axis=1, keepdims=True)
        cols.append(am)
        vals = jnp.where(iota == am, jnp.inf, vals)
    idx_ref[...] = jnp.concatenate(cols, axis=1)


def _knn_pallas(pos, k):
    n = pos.shape[0]
    sq = jnp.sum(pos * pos, axis=1)
    a_q = jnp.concatenate([pos, jnp.zeros((n, 5), jnp.float32)], axis=1)
    b_k = jnp.concatenate([pos.T, jnp.zeros((5, n), jnp.float32)], axis=0)
    return pl.pallas_call(
        functools.partial(_knn_body, k=k),
        grid=(n // _BQ,),
        in_specs=[
            pl.BlockSpec((_BQ, 8), lambda i: (i, 0)),
            pl.BlockSpec((8, n), lambda i: (0, 0)),
            pl.BlockSpec((_BQ, 1), lambda i: (i, 0)),
            pl.BlockSpec((1, n), lambda i: (0, 0)),
        ],
        out_specs=pl.BlockSpec((_BQ, k), lambda i: (i, 0)),
        out_shape=jax.ShapeDtypeStruct((n, k), jnp.int32),
    )(a_q, b_k, sq[:, None], sq[None, :])


def _jacobi_rot(app, aqq, apq):
    zero = apq == 0.0
    tau = (aqq - app) / (2.0 * jnp.where(zero, 1.0, apq))
    t = jnp.sign(tau) / (jnp.abs(tau) + jnp.sqrt(1.0 + tau * tau))
    t = jnp.where(jnp.isnan(t), 0.0, t)
    c = jax.lax.rsqrt(1.0 + t * t)
    s = t * c
    c = jnp.where(zero, 1.0, c)
    s = jnp.where(zero, 0.0, s)
    return c, s


def _smallest_evec(a11, a22, a33, a12, a13, a23):
    v = [[jnp.ones_like(a11), jnp.zeros_like(a11), jnp.zeros_like(a11)],
         [jnp.zeros_like(a11), jnp.ones_like(a11), jnp.zeros_like(a11)],
         [jnp.zeros_like(a11), jnp.zeros_like(a11), jnp.ones_like(a11)]]
    m = {(0, 0): a11, (1, 1): a22, (2, 2): a33,
         (0, 1): a12, (0, 2): a13, (1, 2): a23}
    for _ in range(6):
        for (p, q) in ((0, 1), (0, 2), (1, 2)):
            r = 3 - p - q
            c, s = _jacobi_rot(m[(p, p)], m[(q, q)], m[(p, q)])
            app, aqq, apq = m[(p, p)], m[(q, q)], m[(p, q)]
            apr = m[(min(p, r), max(p, r))]
            aqr = m[(min(q, r), max(q, r))]
            m[(p, p)] = c * c * app - 2.0 * s * c * apq + s * s * aqq
            m[(q, q)] = s * s * app + 2.0 * s * c * apq + c * c * aqq
            m[(p, q)] = jnp.zeros_like(apq)
            m[(min(p, r), max(p, r))] = c * apr - s * aqr
            m[(min(q, r), max(q, r))] = s * apr + c * aqr
            for i in range(3):
                vip, viq = v[i][p], v[i][q]
                v[i][p] = c * vip - s * viq
                v[i][q] = s * vip + c * viq
    d = [m[(0, 0)], m[(1, 1)], m[(2, 2)]]
    c1 = (d[0] <= d[1]) & (d[0] <= d[2])
    c2 = (d[1] <= d[0]) & (d[1] <= d[2])
    pick = lambda row: jnp.where(c1, row[0], jnp.where(c2, row[1], row[2]))
    nx, ny, nz = pick(v[0]), pick(v[1]), pick(v[2])
    inv = jax.lax.rsqrt(nx * nx + ny * ny + nz * nz)
    return nx * inv, ny * inv, nz * inv


def _geom_body(off_ref, g_ref, v0_ref):
    off = off_ref[...]
    ox, oy, oz = off[0], off[1], off[2]
    cx = ox[:K_NORMAL] - jnp.mean(ox[:K_NORMAL], axis=0, keepdims=True)
    cy = oy[:K_NORMAL] - jnp.mean(oy[:K_NORMAL], axis=0, keepdims=True)
    cz = oz[:K_NORMAL] - jnp.mean(oz[:K_NORMAL], axis=0, keepdims=True)
    nx, ny, nz = _smallest_evec(
        jnp.sum(cx * cx, axis=0), jnp.sum(cy * cy, axis=0),
        jnp.sum(cz * cz, axis=0), jnp.sum(cx * cy, axis=0),
        jnp.sum(cx * cz, axis=0), jnp.sum(cy * cz, axis=0))
    cnd = jnp.abs(nz) < 0.9
    tx = jnp.where(cnd, ny, 0.0)
    ty = jnp.where(cnd, -nx, nz)
    tz = jnp.where(cnd, 0.0, -ny)
    tinv = jax.lax.rsqrt(tx * tx + ty * ty + tz * tz)
    xbx, xby, xbz = tx * tinv, ty * tinv, tz * tinv
    ybx = ny * xbz - nz * xby
    yby = nz * xbx - nx * xbz
    ybz = nx * xby - ny * xbx
    x1 = ox * xbx + oy * xby + oz * xbz
    x2 = ox * ybx + oy * yby + oz * ybz
    w = jnp.exp(-(ox * ox + oy * oy + oz * oz) / (KW * KW))
    a11 = jnp.sum(w * x1 * x1, axis=0) + REG
    a22 = jnp.sum(w * x2 * x2, axis=0) + REG
    a12 = jnp.sum(w * x1 * x2, axis=0)
    dinv = 1.0 / (a11 * a22 - a12 * a12)
    i11, i12, i22 = a22 * dinv, -a12 * dinv, a11 * dinv
    g1 = (i11 * x1 + i12 * x2) * w
    g2 = (i12 * x1 + i22 * x2) * w
    g_ref[...] = jnp.stack([g1, g2], axis=0)
    v0_ref[...] = jnp.stack(
        [jnp.stack([jnp.sum(g1 * ox, axis=0), jnp.sum(g1 * oy, axis=0),
                    jnp.sum(g1 * oz, axis=0)], axis=0),
         jnp.stack([jnp.sum(g2 * ox, axis=0), jnp.sum(g2 * oy, axis=0),
                    jnp.sum(g2 * oz, axis=0)], axis=0)], axis=0)


def _geom_pallas(off_t):
    n = off_t.shape[2]
    return pl.pallas_call(
        _geom_body,
        grid=(n // _BN,),
        in_specs=[pl.BlockSpec((3, K_NBRS, _BN), lambda i: (0, 0, i))],
        out_specs=[
            pl.BlockSpec((2, K_NBRS, _BN), lambda i: (0, 0, i)),
            pl.BlockSpec((2, 3, _BN), lambda i: (0, 0, i)),
        ],
        out_shape=[
            jax.ShapeDtypeStruct((2, K_NBRS, n), jnp.float32),
            jax.ShapeDtypeStruct((2, 3, n), jnp.float32),
        ],
    )(off_t)


_SC_NC = 2
_SC_NS = 16
_NW = _SC_NC * _SC_NS


def _sc_chunk(bpw, c):
    sub = bpw
    while sub * c * 4 > 320 * 1024:
        sub //= 2
    return sub, bpw // sub


def _sc_gather(table, idx):
    v, c = table.shape
    b = idx.shape[0]
    bpw = b // _NW
    sub, nsub = _sc_chunk(bpw, c)
    mesh = plsc.VectorSubcoreMesh(core_axis_name="c", subcore_axis_name="s")

    @functools.partial(
        pl.kernel, mesh=mesh,
        out_type=jax.ShapeDtypeStruct((b, c), jnp.float32),
        compiler_params=pltpu.CompilerParams(use_tc_tiling_on_sc=False),
        scratch_types=[
            pltpu.VMEM((bpw,), jnp.int32),
            pltpu.VMEM((sub, c), jnp.float32),
            pltpu.SemaphoreType.DMA,
        ],
    )
    def k(table_hbm, idx_hbm, out_hbm, idx_v, rows_v, sem):
        wid = lax.axis_index("s") * _SC_NC + lax.axis_index("c")
        base = wid * bpw
        pltpu.sync_copy(idx_hbm.at[pl.ds(base, bpw)], idx_v)
        for j in range(nsub):
            pltpu.async_copy(
                table_hbm.at[idx_v.at[pl.ds(j * sub, sub)]], rows_v, sem
            ).wait()
            pltpu.sync_copy(rows_v, out_hbm.at[pl.ds(base + j * sub, sub)])

    return k(table, idx)


def _sc_scatter_add(contrib, idx_dual, nacc, zeros):
    b, c = contrib.shape
    bpc = b // _SC_NS
    nsub, sub = idx_dual.shape[1], idx_dual.shape[2]
    rpt = nacc // _SC_NS
    mesh = plsc.VectorSubcoreMesh(core_axis_name="c", subcore_axis_name="s")

    @functools.partial(
        pl.kernel, mesh=mesh,
        out_type=jax.ShapeDtypeStruct((_SC_NC, nacc, c), jnp.float32),
        compiler_params=pltpu.CompilerParams(use_tc_tiling_on_sc=False),
        scratch_types=[
            pltpu.VMEM((nsub, sub), jnp.int32),
            pltpu.VMEM((sub, c), jnp.float32),
            pltpu.VMEM_SHARED((nacc, c), jnp.float32),
        ],
    )
    def k(contrib_hbm, idx_hbm, zeros_hbm, out_hbm, idx_v, rows_v, acc_sh):
        cid = lax.axis_index("c")
        sid = lax.axis_index("s")
        wid = sid * _SC_NC + cid
        pltpu.sync_copy(zeros_hbm.at[pl.ds(sid * rpt, rpt)],
                        acc_sh.at[pl.ds(sid * rpt, rpt)])
        plsc.subcore_barrier()
        pltpu.sync_copy(idx_hbm.at[wid], idx_v)
        for j in range(nsub):
            pltpu.sync_copy(
                contrib_hbm.at[pl.ds(sid * bpc + j * sub, sub)], rows_v)
            pltpu.sync_copy(rows_v, acc_sh.at[idx_v.at[j]], add=True)
        plsc.subcore_barrier()
        pltpu.sync_copy(acc_sh.at[pl.ds(sid * rpt, rpt)],
                        out_hbm.at[cid, pl.ds(sid * rpt, rpt)])

    return k(contrib, idx_dual, zeros)


def _mm_relu_body(x_ref, w_ref, b_ref, o_ref):
    o_ref[...] = jax.nn.relu(
        jnp.dot(x_ref[...], w_ref[...], preferred_element_type=jnp.float32)
        + b_ref[...])


def _mm_relu(x, w, b):
    n, _ = x.shape
    co = w.shape[1]
    return pl.pallas_call(
        _mm_relu_body,
        out_shape=jax.ShapeDtypeStruct((n, co), jnp.float32),
    )(x, w, b[None, :])


def _pad16(x):
    c = x.shape[-1]
    cp = -(-c // 16) * 16
    if cp == c:
        return x
    return jnp.concatenate(
        [x, jnp.zeros(x.shape[:-1] + (cp - c,), x.dtype)], axis=-1)


def _gather_rows(s, idx):
    n, c = s.shape
    k = idx.shape[1]
    rows = _sc_gather(_pad16(s), idx.reshape(-1))
    return rows.reshape(n, k, -1)[:, :, :c]


def _grad_op(G, idx, s):
    diff = _gather_rows(s, idx) - s[:, None, :]
    return jnp.einsum('nak,nkc->nac', G, diff)


def _div_op(G, idx, v, n):
    k = idx.shape[1]
    contrib = _pad16(jnp.einsum('nak,nac->nkc', G, v).reshape(n * k, -1))
    cp = contrib.shape[-1]
    half = n // 2
    nacc = half + 128
    bpc = (n * k) // _SC_NS
    sub, nsub = _sc_chunk(bpc, cp)
    ch = idx.reshape(_SC_NS, bpc)
    dump = half + (jnp.arange(_SC_NS, dtype=jnp.int32) * 8)[:, None]
    lo = jnp.where(ch < half, ch, dump)
    hi = jnp.where(ch >= half, ch - half, dump)
    idx_dual = jnp.stack([lo, hi], axis=1).reshape(_NW, nsub, sub)
    parts = _sc_scatter_add(contrib, idx_dual, nacc,
                            jnp.zeros((nacc, cp), jnp.float32))
    d = jnp.concatenate([parts[0, :half], parts[1, :half]], axis=0)
    c = v.shape[-1]
    return d[:, :c] - jnp.einsum('nak,nac->nc', G, v)


def _delta_conv(x, v, G, idx, n, Ws, bs, Wv):
    vn = jnp.sqrt(jnp.sum(v ** 2, axis=1) + 1e-8)
    dv = _div_op(G, idx, v, n)
    h = _mm_relu(jnp.concatenate([x, vn, dv], axis=-1), Ws, bs)
    if Wv is None:
        return h, None
    v_new = jnp.einsum('nac,co->nao', v, Wv) + _grad_op(G, idx, h)
    return h, v_new


def kernel(points, x1_dummy, als_ppoints, Ws1, bs1, Wv1, Ws2, bs2, Wv2, Ws3, bs3):
    pos = jnp.concatenate([points, als_ppoints], axis=0)
    n = pos.shape[0]
    idx = _knn_pallas(pos, K_NBRS)
    off = _gather_rows(pos, idx) - pos[:, None, :]
    g_t, v0_t = _geom_pallas(off.transpose(2, 1, 0))
    G = g_t.transpose(2, 0, 1)
    v = v0_t.transpose(2, 0, 1)
    x = pos - pos.mean(axis=0, keepdims=True)
    out = []
    x, v = _delta_conv(x, v, G, idx, n, Ws1, bs1, Wv1)
    out.append(x)
    x, v = _delta_conv(x, v, G, idx, n, Ws2, bs2, Wv2)
    out.append(x)
    x, _ = _delta_conv(x, v, G, idx, n, Ws3, bs3, None)
    out.append(x)
    return tuple(out)
